# baseline (speedup 1.0000x reference)
import jax
import jax.numpy as jnp
from jax.experimental import pallas as pl
from jax.experimental.pallas import tpu as pltpu

_NUM_GRAPHS = 32
_NPG = 64
_EPG = 256
_H = 128
_KS = 8


def _fused_gnn_kernel(x_ref, de_ref, se_ref,
                      w1i0_ref, w1j0_ref, b10_ref, w20_ref, b20_ref,
                      w30_ref, b30_ref,
                      w1i1_ref, w1j1_ref, b11_ref, w21_ref, b21_ref,
                      w31_ref, b31_ref,
                      w1i2_ref, w1j2_ref, b12_ref, w22_ref, b22_ref,
                      w32_ref, b32_ref,
                      w1i3_ref, w1j3_ref, b13_ref, w23_ref, b23_ref,
                      w33_ref, b33_ref,
                      hw_ref, hb_ref, o_ref, acc_ref):
    k = pl.program_id(0)

    def _bdot(a, b):
        return jnp.dot(a.astype(jnp.bfloat16), b.astype(jnp.bfloat16),
                       preferred_element_type=jnp.float32)

    xb = x_ref[...].astype(jnp.bfloat16)
    pi = _bdot(xb, w1i0_ref[...])
    pj = _bdot(xb, w1j0_ref[...])

    @pl.when(k == 0)
    def _():
        acc_ref[:, :_H] = pi
        acc_ref[:, _H:] = pj

    @pl.when(k > 0)
    def _():
        acc_ref[:, :_H] += pi
        acc_ref[:, _H:] += pj

    @pl.when(k == _KS - 1)
    def _():
        n = _NUM_GRAPHS * _NPG
        iota_e = jax.lax.broadcasted_iota(jnp.int32, (_NPG, _EPG), 0)
        dts, sts, degs = [], [], []
        for g in range(_NUM_GRAPHS):
            dlg = de_ref[0][:, g * _EPG:(g + 1) * _EPG] % _NPG
            slg = se_ref[0][:, g * _EPG:(g + 1) * _EPG] % _NPG
            dts.append((iota_e == dlg).astype(jnp.bfloat16))
            sts.append((iota_e == slg).astype(jnp.bfloat16))
            degs.append(jnp.sum(dts[g].astype(jnp.float32), axis=1,
                                keepdims=True))
        deg = jnp.concatenate(degs, axis=0)

        def _gather(onehot_t, v):
            return jax.lax.dot_general(
                onehot_t, v.astype(jnp.bfloat16),
                (((0,), (0,)), ((), ())),
                preferred_element_type=jnp.float32)

        def message_pass(hi, hj, w2, b2, w3, b3, relu_out):
            h1 = jnp.maximum(jnp.concatenate(
                [_gather(dts[g], hi[g * _NPG:(g + 1) * _NPG])
                 + _gather(sts[g], hj[g * _NPG:(g + 1) * _NPG])
                 for g in range(_NUM_GRAPHS)], axis=0), 0.0)
            h2 = jnp.maximum(_bdot(h1, w2) + b2, 0.0)
            agg = jnp.concatenate(
                [_bdot(dts[g], h2[g * _EPG:(g + 1) * _EPG])
                 for g in range(_NUM_GRAPHS)], axis=0)
            out = _bdot(agg, w3) + deg * b3
            return jnp.maximum(out, 0.0) if relu_out else out

        h = message_pass(acc_ref[:, :_H] + b10_ref[...], acc_ref[:, _H:],
                         w20_ref[...], b20_ref[...], w30_ref[...],
                         b30_ref[...], relu_out=True)
        layers = [
            (w1i1_ref, w1j1_ref, b11_ref, w21_ref, b21_ref, w31_ref, b31_ref),
            (w1i2_ref, w1j2_ref, b12_ref, w22_ref, b22_ref, w32_ref, b32_ref),
            (w1i3_ref, w1j3_ref, b13_ref, w23_ref, b23_ref, w33_ref, b33_ref),
        ]
        for li, (w1i, w1j, b1, w2, b2, w3, b3) in enumerate(layers):
            hi = _bdot(h, w1i[...]) + b1[...]
            hj = _bdot(h, w1j[...])
            h = message_pass(hi, hj, w2[...], b2[...], w3[...], b3[...],
                             relu_out=(li < 2))

        pooled = h.reshape(_NUM_GRAPHS, _NPG, _H).mean(axis=1)
        out = (jnp.dot(pooled, hw_ref[...],
                       preferred_element_type=jnp.float32)
               + hb_ref[...])
        o_ref[...] = out.reshape(1, _NUM_GRAPHS,
                                 out.shape[-1]).astype(o_ref.dtype)


def kernel(x, edge_index, batch,
           l0_w1i, l0_w1j, l0_b1, l0_w2, l0_b2, l0_w3, l0_b3,
           l1_w1i, l1_w1j, l1_b1, l1_w2, l1_b2, l1_w3, l1_b3,
           l2_w1i, l2_w1j, l2_b1, l2_w2, l2_b2, l2_w3, l2_b3,
           l3_w1i, l3_w1j, l3_b1, l3_w2, l3_b2, l3_w3, l3_b3,
           head_w, head_b):
    N, F = x.shape
    C = head_w.shape[1]
    ne = edge_index.shape[1]
    de = edge_index[1].reshape(1, 1, ne)
    se = edge_index[0].reshape(1, 1, ne)

    inv = lambda k: (0, 0)
    inv3 = lambda k: (0, 0, 0)
    wspecs = []
    for _ in range(4):
        wspecs += [
            pl.BlockSpec((_H, _H), inv), pl.BlockSpec((_H, _H), inv),
            pl.BlockSpec((1, _H), inv), pl.BlockSpec((_H, _H), inv),
            pl.BlockSpec((1, _H), inv), pl.BlockSpec((_H, _H), inv),
            pl.BlockSpec((1, _H), inv),
        ]
    wspecs[0] = pl.BlockSpec((F // _KS, _H), lambda k: (k, 0))
    wspecs[1] = pl.BlockSpec((F // _KS, _H), lambda k: (k, 0))

    out = pl.pallas_call(
        _fused_gnn_kernel,
        out_shape=jax.ShapeDtypeStruct((1, _NUM_GRAPHS, C), jnp.float32),
        grid=(_KS,),
        in_specs=[
            pl.BlockSpec((N, F // _KS), lambda k: (0, k)),
            pl.BlockSpec((1, 1, ne), inv3),
            pl.BlockSpec((1, 1, ne), inv3),
        ] + wspecs + [
            pl.BlockSpec((_H, C), inv),
            pl.BlockSpec((1, C), inv),
        ],
        out_specs=pl.BlockSpec((1, _NUM_GRAPHS, C), inv3),
        scratch_shapes=[pltpu.VMEM((N, 2 * _H), jnp.float32)],
        compiler_params=pltpu.CompilerParams(
            dimension_semantics=("arbitrary",)),
    )(x, de, se,
      l0_w1i, l0_w1j, l0_b1, l0_w2, l0_b2, l0_w3, l0_b3,
      l1_w1i, l1_w1j, l1_b1, l1_w2, l1_b2, l1_w3, l1_b3,
      l2_w1i, l2_w1j, l2_b1, l2_w2, l2_b2, l2_w3, l2_b3,
      l3_w1i, l3_w1j, l3_b1, l3_w2, l3_b2, l3_w3, l3_b3,
      head_w, head_b)
    return out.reshape(_NUM_GRAPHS, C)


# speedup vs baseline: 1.1048x; 1.1048x over previous
import jax
import jax.numpy as jnp
from jax.experimental import pallas as pl
from jax.experimental.pallas import tpu as pltpu

_NUM_GRAPHS = 32
_NPG = 64
_EPG = 256
_H = 128
_KS = 4


def _fused_gnn_kernel(x_ref, de_ref, se_ref,
                      w1i0_ref, w1j0_ref, b10_ref, w20_ref, b20_ref,
                      w30_ref, b30_ref,
                      w1i1_ref, w1j1_ref, b11_ref, w21_ref, b21_ref,
                      w31_ref, b31_ref,
                      w1i2_ref, w1j2_ref, b12_ref, w22_ref, b22_ref,
                      w32_ref, b32_ref,
                      w1i3_ref, w1j3_ref, b13_ref, w23_ref, b23_ref,
                      w33_ref, b33_ref,
                      hw_ref, hb_ref, o_ref, acc_ref):
    k = pl.program_id(0)

    def _bdot(a, b):
        return jnp.dot(a.astype(jnp.bfloat16), b.astype(jnp.bfloat16),
                       preferred_element_type=jnp.float32)

    xb = x_ref[...].astype(jnp.bfloat16)
    pi = _bdot(xb, w1i0_ref[...])
    pj = _bdot(xb, w1j0_ref[...])

    @pl.when(k == 0)
    def _():
        acc_ref[:, :_H] = pi
        acc_ref[:, _H:] = pj

    @pl.when(k > 0)
    def _():
        acc_ref[:, :_H] += pi
        acc_ref[:, _H:] += pj

    @pl.when(k == _KS - 1)
    def _():
        n = _NUM_GRAPHS * _NPG
        iota_e = jax.lax.broadcasted_iota(jnp.int32, (_NPG, _EPG), 0)
        dts, sts, degs = [], [], []
        for g in range(_NUM_GRAPHS):
            dlg = de_ref[0][:, g * _EPG:(g + 1) * _EPG] % _NPG
            slg = se_ref[0][:, g * _EPG:(g + 1) * _EPG] % _NPG
            dts.append((iota_e == dlg).astype(jnp.bfloat16))
            sts.append((iota_e == slg).astype(jnp.bfloat16))
            degs.append(jnp.sum(dts[g].astype(jnp.float32), axis=1,
                                keepdims=True))
        deg = jnp.concatenate(degs, axis=0)

        def _gather(onehot_t, v):
            return jax.lax.dot_general(
                onehot_t, v.astype(jnp.bfloat16),
                (((0,), (0,)), ((), ())),
                preferred_element_type=jnp.float32)

        def message_pass(hi, hj, w2, b2, w3, b3, relu_out):
            h1 = jnp.maximum(jnp.concatenate(
                [_gather(dts[g], hi[g * _NPG:(g + 1) * _NPG])
                 + _gather(sts[g], hj[g * _NPG:(g + 1) * _NPG])
                 for g in range(_NUM_GRAPHS)], axis=0), 0.0)
            h2 = jnp.maximum(_bdot(h1, w2) + b2, 0.0)
            agg = jnp.concatenate(
                [_bdot(dts[g], h2[g * _EPG:(g + 1) * _EPG])
                 for g in range(_NUM_GRAPHS)], axis=0)
            out = _bdot(agg, w3) + deg * b3
            return jnp.maximum(out, 0.0) if relu_out else out

        h = message_pass(acc_ref[:, :_H] + b10_ref[...], acc_ref[:, _H:],
                         w20_ref[...], b20_ref[...], w30_ref[...],
                         b30_ref[...], relu_out=True)
        layers = [
            (w1i1_ref, w1j1_ref, b11_ref, w21_ref, b21_ref, w31_ref, b31_ref),
            (w1i2_ref, w1j2_ref, b12_ref, w22_ref, b22_ref, w32_ref, b32_ref),
            (w1i3_ref, w1j3_ref, b13_ref, w23_ref, b23_ref, w33_ref, b33_ref),
        ]
        for li, (w1i, w1j, b1, w2, b2, w3, b3) in enumerate(layers):
            hi = _bdot(h, w1i[...]) + b1[...]
            hj = _bdot(h, w1j[...])
            h = message_pass(hi, hj, w2[...], b2[...], w3[...], b3[...],
                             relu_out=(li < 2))

        pooled = h.reshape(_NUM_GRAPHS, _NPG, _H).mean(axis=1)
        out = (jnp.dot(pooled, hw_ref[...],
                       preferred_element_type=jnp.float32)
               + hb_ref[...])
        o_ref[...] = out.reshape(1, _NUM_GRAPHS,
                                 out.shape[-1]).astype(o_ref.dtype)


def kernel(x, edge_index, batch,
           l0_w1i, l0_w1j, l0_b1, l0_w2, l0_b2, l0_w3, l0_b3,
           l1_w1i, l1_w1j, l1_b1, l1_w2, l1_b2, l1_w3, l1_b3,
           l2_w1i, l2_w1j, l2_b1, l2_w2, l2_b2, l2_w3, l2_b3,
           l3_w1i, l3_w1j, l3_b1, l3_w2, l3_b2, l3_w3, l3_b3,
           head_w, head_b):
    N, F = x.shape
    C = head_w.shape[1]
    ne = edge_index.shape[1]
    de = edge_index[1].reshape(1, 1, ne)
    se = edge_index[0].reshape(1, 1, ne)

    inv = lambda k: (0, 0)
    inv3 = lambda k: (0, 0, 0)
    wspecs = []
    for _ in range(4):
        wspecs += [
            pl.BlockSpec((_H, _H), inv), pl.BlockSpec((_H, _H), inv),
            pl.BlockSpec((1, _H), inv), pl.BlockSpec((_H, _H), inv),
            pl.BlockSpec((1, _H), inv), pl.BlockSpec((_H, _H), inv),
            pl.BlockSpec((1, _H), inv),
        ]
    wspecs[0] = pl.BlockSpec((F // _KS, _H), lambda k: (k, 0))
    wspecs[1] = pl.BlockSpec((F // _KS, _H), lambda k: (k, 0))

    out = pl.pallas_call(
        _fused_gnn_kernel,
        out_shape=jax.ShapeDtypeStruct((1, _NUM_GRAPHS, C), jnp.float32),
        grid=(_KS,),
        in_specs=[
            pl.BlockSpec((N, F // _KS), lambda k: (0, k)),
            pl.BlockSpec((1, 1, ne), inv3),
            pl.BlockSpec((1, 1, ne), inv3),
        ] + wspecs + [
            pl.BlockSpec((_H, C), inv),
            pl.BlockSpec((1, C), inv),
        ],
        out_specs=pl.BlockSpec((1, _NUM_GRAPHS, C), inv3),
        scratch_shapes=[pltpu.VMEM((N, 2 * _H), jnp.float32)],
        compiler_params=pltpu.CompilerParams(
            dimension_semantics=("arbitrary",)),
    )(x, de, se,
      l0_w1i, l0_w1j, l0_b1, l0_w2, l0_b2, l0_w3, l0_b3,
      l1_w1i, l1_w1j, l1_b1, l1_w2, l1_b2, l1_w3, l1_b3,
      l2_w1i, l2_w1j, l2_b1, l2_w2, l2_b2, l2_w3, l2_b3,
      l3_w1i, l3_w1j, l3_b1, l3_w2, l3_b2, l3_w3, l3_b3,
      head_w, head_b)
    return out.reshape(_NUM_GRAPHS, C)
